# baseline (speedup 1.0000x reference)
import functools

import jax
import jax.numpy as jnp
from jax import lax
from jax.experimental import pallas as pl
from jax.experimental.pallas import tpu as pltpu
from jax.experimental.pallas import tpu_sc as plsc

B, F, D = 16384, 26, 32
H = D // 2
FP = 32

_info = plsc.get_sparse_core_info()
_NC, _NS = _info.num_cores, _info.num_subcores
NW = _NC * _NS
C = B // NW
S = 128
NSUB = C // S
FS = S * F
NB = B // S
assert C % S == 0 and B % NW == 0 and S % 16 == 0


def _unpack_bf16(row):
  x = plsc.bitcast(row, jnp.int32)
  a = plsc.bitcast(x << 16, jnp.float32)
  b = plsc.bitcast((x >> 16) << 16, jnp.float32)
  return a, b


def _tile_flat(x):
  xp = jnp.pad(x, ((0, 0), (0, FP - F)))
  return xp.reshape(NB, S, FP // 8, 8).transpose(2, 0, 3, 1).reshape(-1)


def _sc_forward(user_ids, item_ids, ufi, ufv, ifi, ifv, U, I, UF, IF):
  mesh = plsc.VectorSubcoreMesh(core_axis_name="c", subcore_axis_name="s")

  @functools.partial(
      pl.kernel,
      mesh=mesh,
      compiler_params=pltpu.CompilerParams(use_tc_tiling_on_sc=False,
                                           needs_layout_passes=False),
      out_type=jax.ShapeDtypeStruct((B,), jnp.float32),
      scratch_types=[
          pltpu.VMEM((S,), jnp.int32),
          pltpu.VMEM((S,), jnp.int32),
          pltpu.VMEM((FP * S,), jnp.int32),
          pltpu.VMEM((FP * S,), jnp.float32),
          pltpu.VMEM((FP * S,), jnp.int32),
          pltpu.VMEM((FP * S,), jnp.float32),
          pltpu.VMEM((S, D), jnp.bfloat16),
          pltpu.VMEM((S, D), jnp.bfloat16),
          pltpu.VMEM((FS, D), jnp.bfloat16),
          pltpu.VMEM((FS, D), jnp.bfloat16),
          pltpu.VMEM((S,), jnp.float32),
          pltpu.SemaphoreType.DMA,
      ],
  )
  def k(uid_h, iid_h, ufi_h, ufv_h, ifi_h, ifv_h, U_h, I_h, UF_h, IF_h,
        out_h, uids_v, iids_v, ufi_v, ufv_v, ifi_v, ifv_v,
        urows_v, irows_v, ufrows_v, ifrows_v, out_v, sem):
    wid = lax.axis_index("s") * _NC + lax.axis_index("c")
    lane_iota = lax.iota(jnp.int32, 16)
    wlo_idx = lane_iota * S
    whi_idx = jnp.minimum(lane_iota + 16, F - 1) * S

    def sub(j, carry):
      base = wid * C + j * S
      blk = wid * NSUB + j
      pltpu.sync_copy(uid_h.at[pl.ds(base, S)], uids_v)
      pltpu.sync_copy(iid_h.at[pl.ds(base, S)], iids_v)
      for a in range(FP // 8):
        off = (a * NB + blk) * 8 * S
        rows = pl.ds(a * 8 * S, 8 * S)
        pltpu.sync_copy(ufi_h.at[pl.ds(off, 8 * S)], ufi_v.at[rows])
        pltpu.sync_copy(ufv_h.at[pl.ds(off, 8 * S)], ufv_v.at[rows])
        pltpu.sync_copy(ifi_h.at[pl.ds(off, 8 * S)], ifi_v.at[rows])
        pltpu.sync_copy(ifv_h.at[pl.ds(off, 8 * S)], ifv_v.at[rows])
      cps = [pltpu.async_copy(U_h.at[uids_v], urows_v, sem),
             pltpu.async_copy(I_h.at[iids_v], irows_v, sem)]
      for f in range(F):
        sl = pl.ds(f * S, S)
        cps.append(pltpu.async_copy(UF_h.at[ufi_v.at[sl]], ufrows_v.at[sl], sem))
        cps.append(pltpu.async_copy(IF_h.at[ifi_v.at[sl]], ifrows_v.at[sl], sem))
      for cp in cps:
        cp.wait()

      def group(bg, carry2):
        def lane(l, acc):
          b = bg * 16 + l
          bvec = jnp.zeros((16,), jnp.int32) + b
          u0, u1 = _unpack_bf16(urows_v[b, :])
          i0, i1 = _unpack_bf16(irows_v[b, :])
          uw0 = plsc.load_gather(ufv_v, [wlo_idx + bvec])
          uw1 = plsc.load_gather(ufv_v, [whi_idx + bvec])
          iw0 = plsc.load_gather(ifv_v, [wlo_idx + bvec])
          iw1 = plsc.load_gather(ifv_v, [whi_idx + bvec])
          for f in range(F):
            p = f * S + b
            wu = uw0[f] if f < 16 else uw1[f - 16]
            ua, ub = _unpack_bf16(ufrows_v[p, :])
            u0 = u0 + ua * wu
            u1 = u1 + ub * wu
            wi = iw0[f] if f < 16 else iw1[f - 16]
            ia, ib = _unpack_bf16(ifrows_v[p, :])
            i0 = i0 + ia * wi
            i1 = i1 + ib * wi
          prod = u0 * i0 + u1 * i1
          for sh in (8, 4, 2, 1):
            prod = prod + prod[lane_iota ^ sh]
          return jnp.where(lane_iota == l, prod, acc)

        acc = lax.fori_loop(0, 16, lane, jnp.zeros((16,), jnp.float32))
        out_v[pl.ds(bg * 16, 16)] = acc
        return carry2

      lax.fori_loop(0, S // 16, group, 0)
      pltpu.sync_copy(out_v, out_h.at[pl.ds(base, S)])
      return carry

    lax.fori_loop(0, NSUB, sub, 0)

  return k(user_ids, item_ids, ufi, ufv, ifi, ifv, U, I, UF, IF)


def kernel(user_ids, item_ids, user_feature_indices, user_feature_values,
           item_feature_indices, item_feature_values, U, I, UF, IF):
  return _sc_forward(
      user_ids.astype(jnp.int32),
      item_ids.astype(jnp.int32),
      _tile_flat(user_feature_indices.astype(jnp.int32)),
      _tile_flat(user_feature_values),
      _tile_flat(item_feature_indices.astype(jnp.int32)),
      _tile_flat(item_feature_values),
      U.astype(jnp.bfloat16), I.astype(jnp.bfloat16),
      UF.astype(jnp.bfloat16), IF.astype(jnp.bfloat16))


# speedup vs baseline: 1.1985x; 1.1985x over previous
import functools

import jax
import jax.numpy as jnp
from jax import lax
from jax.experimental import pallas as pl
from jax.experimental.pallas import tpu as pltpu
from jax.experimental.pallas import tpu_sc as plsc

B, F, D = 16384, 26, 32
H = D // 2
FP = 32

_info = plsc.get_sparse_core_info()
_NC, _NS = _info.num_cores, _info.num_subcores
NW = _NC * _NS
C = B // NW
S = 128
NSUB = C // S
FS = S * F
NB = B // S
assert C % S == 0 and B % NW == 0 and S % 16 == 0


def _unpack_bf16(row):
  x = plsc.bitcast(row, jnp.int32)
  a = plsc.bitcast(x << 16, jnp.float32)
  b = plsc.bitcast((x >> 16) << 16, jnp.float32)
  return a, b


def _tile_flat(x):
  xp = jnp.pad(x, ((0, 0), (0, FP - F)))
  return xp.reshape(NB, S, FP // 8, 8).transpose(2, 0, 3, 1).reshape(-1)


def _sc_forward(user_ids, item_ids, ufi, ufv, ifi, ifv, U, I, UF, IF):
  mesh = plsc.VectorSubcoreMesh(core_axis_name="c", subcore_axis_name="s")

  @functools.partial(
      pl.kernel,
      mesh=mesh,
      compiler_params=pltpu.CompilerParams(use_tc_tiling_on_sc=False,
                                           needs_layout_passes=False),
      out_type=jax.ShapeDtypeStruct((B,), jnp.float32),
      scratch_types=[
          pltpu.VMEM((S,), jnp.int32),
          pltpu.VMEM((S,), jnp.int32),
          pltpu.VMEM((F * S,), jnp.int32),
          pltpu.VMEM((F * S,), jnp.float32),
          pltpu.VMEM((F * S,), jnp.int32),
          pltpu.VMEM((F * S,), jnp.float32),
          pltpu.VMEM((S, D), jnp.float32),
          pltpu.VMEM((S, D), jnp.float32),
          pltpu.VMEM((FS, D), jnp.bfloat16),
          pltpu.VMEM((FS, D), jnp.bfloat16),
          pltpu.VMEM((S,), jnp.float32),
          pltpu.SemaphoreType.DMA,
      ],
  )
  def k(uid_h, iid_h, ufi_h, ufv_h, ifi_h, ifv_h, U_h, I_h, UF_h, IF_h,
        out_h, uids_v, iids_v, ufi_v, ufv_v, ifi_v, ifv_v,
        urows_v, irows_v, ufrows_v, ifrows_v, out_v, sem):
    wid = lax.axis_index("s") * _NC + lax.axis_index("c")
    lane_iota = lax.iota(jnp.int32, 16)
    wlo_idx = lane_iota * S
    whi_idx = jnp.minimum(lane_iota + 16, F - 1) * S

    def sub(j, carry):
      base = wid * C + j * S
      blk = wid * NSUB + j
      pltpu.sync_copy(uid_h.at[pl.ds(base, S)], uids_v)
      pltpu.sync_copy(iid_h.at[pl.ds(base, S)], iids_v)
      for a in range(FP // 8):
        n = 8 * S if a < 3 else (F - 24) * S
        off = (a * NB + blk) * 8 * S
        rows = pl.ds(a * 8 * S, n)
        pltpu.sync_copy(ufi_h.at[pl.ds(off, n)], ufi_v.at[rows])
        pltpu.sync_copy(ufv_h.at[pl.ds(off, n)], ufv_v.at[rows])
        pltpu.sync_copy(ifi_h.at[pl.ds(off, n)], ifi_v.at[rows])
        pltpu.sync_copy(ifv_h.at[pl.ds(off, n)], ifv_v.at[rows])
      cps = [pltpu.async_copy(U_h.at[uids_v], urows_v, sem),
             pltpu.async_copy(I_h.at[iids_v], irows_v, sem)]
      for f in range(F):
        sl = pl.ds(f * S, S)
        cps.append(pltpu.async_copy(UF_h.at[ufi_v.at[sl]], ufrows_v.at[sl], sem))
        cps.append(pltpu.async_copy(IF_h.at[ifi_v.at[sl]], ifrows_v.at[sl], sem))
      for cp in cps:
        cp.wait()

      def group(bg, carry2):
        def lane(l, acc):
          b = bg * 16 + l
          bvec = jnp.zeros((16,), jnp.int32) + b
          u0 = urows_v[b, 0:H]
          u1 = urows_v[b, H:D]
          i0 = irows_v[b, 0:H]
          i1 = irows_v[b, H:D]
          uw0 = plsc.load_gather(ufv_v, [wlo_idx + bvec])
          uw1 = plsc.load_gather(ufv_v, [whi_idx + bvec])
          iw0 = plsc.load_gather(ifv_v, [wlo_idx + bvec])
          iw1 = plsc.load_gather(ifv_v, [whi_idx + bvec])
          for f in range(F):
            p = f * S + b
            wu = uw0[f] if f < 16 else uw1[f - 16]
            ua, ub = _unpack_bf16(ufrows_v[p, :])
            u0 = u0 + ua * wu
            u1 = u1 + ub * wu
            wi = iw0[f] if f < 16 else iw1[f - 16]
            ia, ib = _unpack_bf16(ifrows_v[p, :])
            i0 = i0 + ia * wi
            i1 = i1 + ib * wi
          prod = u0 * i0 + u1 * i1
          for sh in (8, 4, 2, 1):
            prod = prod + prod[lane_iota ^ sh]
          return jnp.where(lane_iota == l, prod, acc)

        acc = lax.fori_loop(0, 16, lane, jnp.zeros((16,), jnp.float32))
        out_v[pl.ds(bg * 16, 16)] = acc
        return carry2

      lax.fori_loop(0, S // 16, group, 0)
      pltpu.sync_copy(out_v, out_h.at[pl.ds(base, S)])
      return carry

    lax.fori_loop(0, NSUB, sub, 0)

  return k(user_ids, item_ids, ufi, ufv, ifi, ifv, U, I, UF, IF)


def kernel(user_ids, item_ids, user_feature_indices, user_feature_values,
           item_feature_indices, item_feature_values, U, I, UF, IF):
  return _sc_forward(
      user_ids.astype(jnp.int32),
      item_ids.astype(jnp.int32),
      _tile_flat(user_feature_indices.astype(jnp.int32)),
      _tile_flat(user_feature_values),
      _tile_flat(item_feature_indices.astype(jnp.int32)),
      _tile_flat(item_feature_values),
      U, I, UF.astype(jnp.bfloat16), IF.astype(jnp.bfloat16))
